# Initial kernel scaffold; baseline (speedup 1.0000x reference)
import jax
import jax.numpy as jnp
from jax.experimental import pallas as pl


def kernel(xv, adj_pos, adj_neg, is_train):
    raise NotImplementedError("write your pallas kernel here")


# speedup vs baseline: 265.9936x; 265.9936x over previous
import jax
import jax.numpy as jnp
from jax import lax
from jax.experimental import pallas as pl
from jax.experimental.pallas import tpu as pltpu
from jax.experimental.pallas import tpu_sc as plsc

P = 3.0
A = 10.0
N_NODES = 100000
N_SEG = 100000
E_EDGES = 3200000

ROWS = 784
NPAD = ROWS * 128
NC = 2
NS = 16
NW = NC * NS
EPW = E_EDGES // NW
CHUNK = 2000
NCH = EPW // CHUNK
ZPT = NPAD // NS


def _tables_body(x_ref, vp_ref, wp_ref, vn_ref, wn_ref):
    x = x_ref[...]
    wp = jnp.exp(P * x)
    xn = 1.0 - x
    wn = jnp.exp(P * xn)
    vp_ref[...] = x * wp
    wp_ref[...] = wp
    vn_ref[...] = xn * wn
    wn_ref[...] = wn


def _make_tables(xp):
    shp = jax.ShapeDtypeStruct((ROWS, 128), jnp.float32)
    return pl.pallas_call(_tables_body, out_shape=(shp, shp, shp, shp))(xp)


def _sc_body(vp, wp, vn, wn, zeros_hbm, adj_pos, adj_neg, out,
             tvp, twp, tvn, twn, acc_num, acc_dom,
             idx_buf, seg_buf, val_buf, wt_buf, sem):
    cid = lax.axis_index("c")
    sid = lax.axis_index("s")
    wid = cid * NS + sid
    sl = pl.ds(sid * ZPT, ZPT)
    sbase = sid * ZPT
    pltpu.sync_copy(vp.at[sl], tvp.at[sl])
    pltpu.sync_copy(wp.at[sl], twp.at[sl])
    pltpu.sync_copy(vn.at[sl], tvn.at[sl])
    pltpu.sync_copy(wn.at[sl], twn.at[sl])
    pltpu.sync_copy(zeros_hbm.at[sl], acc_num.at[sl])
    pltpu.sync_copy(zeros_hbm.at[sl], acc_dom.at[sl])
    plsc.subcore_barrier()

    def run_edges(adj, vtbl, wtbl):
        base0 = wid * EPW

        def chunk(i, carry):
            b = base0 + i * CHUNK
            pltpu.sync_copy(adj.at[pl.ds(E_EDGES + b, CHUNK)], idx_buf)
            pltpu.sync_copy(adj.at[pl.ds(b, CHUNK)], seg_buf)
            pltpu.async_copy(vtbl.at[idx_buf], val_buf, sem).wait()
            pltpu.async_copy(wtbl.at[idx_buf], wt_buf, sem).wait()
            pltpu.sync_copy(val_buf, acc_num.at[seg_buf], add=True)
            pltpu.sync_copy(wt_buf, acc_dom.at[seg_buf], add=True)
            return carry

        lax.fori_loop(0, NCH, chunk, 0)

    run_edges(adj_pos, tvp, twp)
    run_edges(adj_neg, tvn, twn)
    plsc.subcore_barrier()
    pltpu.sync_copy(acc_num.at[sl], out.at[pl.ds((cid * 2 + 0) * NPAD + sbase, ZPT)])
    pltpu.sync_copy(acc_dom.at[sl], out.at[pl.ds((cid * 2 + 1) * NPAD + sbase, ZPT)])


def _segment_accumulate(vp, wp, vn, wn, zeros, adj_pos, adj_neg):
    mesh = plsc.VectorSubcoreMesh(
        core_axis_name="c", subcore_axis_name="s", num_cores=NC, num_subcores=NS)
    f = pl.kernel(
        _sc_body,
        out_type=jax.ShapeDtypeStruct((NC * 2 * NPAD,), jnp.float32),
        mesh=mesh,
        scratch_types=[
            pltpu.VMEM_SHARED((NPAD,), jnp.float32),
            pltpu.VMEM_SHARED((NPAD,), jnp.float32),
            pltpu.VMEM_SHARED((NPAD,), jnp.float32),
            pltpu.VMEM_SHARED((NPAD,), jnp.float32),
            pltpu.VMEM_SHARED((NPAD,), jnp.float32),
            pltpu.VMEM_SHARED((NPAD,), jnp.float32),
            pltpu.VMEM((CHUNK,), jnp.int32),
            pltpu.VMEM((CHUNK,), jnp.int32),
            pltpu.VMEM((CHUNK,), jnp.float32),
            pltpu.VMEM((CHUNK,), jnp.float32),
            pltpu.SemaphoreType.DMA,
        ],
    )
    return f(vp, wp, vn, wn, zeros, adj_pos, adj_neg)


def _loss_body(p_ref, o_ref):
    p = p_ref[...]
    num = p[0, 0] + p[1, 0]
    dom = p[0, 1] + p[1, 1]
    r = num / dom
    sm = 1.0 / (1.0 + jnp.exp(A * (0.5 - r))) + 0.05
    lg = jnp.log(sm)
    row = lax.broadcasted_iota(jnp.int32, (ROWS, 128), 0)
    col = lax.broadcasted_iota(jnp.int32, (ROWS, 128), 1)
    valid = (row * 128 + col) < N_SEG
    o_ref[0, 0] = -jnp.sum(jnp.where(valid, lg, 0.0))


def _finish(parts):
    return pl.pallas_call(
        _loss_body,
        out_shape=jax.ShapeDtypeStruct((1, 1), jnp.float32),
        out_specs=pl.BlockSpec(memory_space=pltpu.SMEM),
    )(parts)


def kernel(xv, adj_pos, adj_neg, is_train):
    xf = xv.reshape(-1).astype(jnp.float32)
    xp = jnp.pad(xf, (0, NPAD - N_NODES)).reshape(ROWS, 128)
    vp, wp, vn, wn = _make_tables(xp)
    zeros = jnp.zeros((NPAD,), jnp.float32)
    parts = _segment_accumulate(
        vp.reshape(NPAD), wp.reshape(NPAD), vn.reshape(NPAD), wn.reshape(NPAD),
        zeros, adj_pos.reshape(-1), adj_neg.reshape(-1))
    loss = _finish(parts.reshape(NC, 2, ROWS, 128))
    return loss[0, 0]


# speedup vs baseline: 371.1789x; 1.3954x over previous
import jax
import jax.numpy as jnp
from jax import lax
from jax.experimental import pallas as pl
from jax.experimental.pallas import tpu as pltpu
from jax.experimental.pallas import tpu_sc as plsc

P = 3.0
A = 10.0
N_NODES = 100000
N_SEG = 100000
E_EDGES = 3200000

ROWS = 784
NPAD = ROWS * 128
NC = 2
NS = 16
NW = NC * NS
EPW = E_EDGES // NW
CHUNK = 2000
NCH = EPW // CHUNK
ZPT = NPAD // NS


def _tables_body(x_ref, vp_ref, wp_ref, vn_ref, wn_ref):
    x = x_ref[...]
    wp = jnp.exp(P * x)
    xn = 1.0 - x
    wn = jnp.exp(P * xn)
    vp_ref[...] = x * wp
    wp_ref[...] = wp
    vn_ref[...] = xn * wn
    wn_ref[...] = wn


def _make_tables(xp):
    shp = jax.ShapeDtypeStruct((ROWS, 128), jnp.float32)
    return pl.pallas_call(_tables_body, out_shape=(shp, shp, shp, shp))(xp)


def _sc_body(vp, wp, vn, wn, zeros_hbm, adj_pos, adj_neg, out,
             tvp, twp, tvn, twn, acc_num, acc_dom,
             idx_a, seg_a, val_a, wt_a, idx_b, seg_b, val_b, wt_b,
             lsem, gsem_v, gsem_w, ssem):
    cid = lax.axis_index("c")
    sid = lax.axis_index("s")
    wid = cid * NS + sid
    sl = pl.ds(sid * ZPT, ZPT)
    sbase = sid * ZPT
    pltpu.sync_copy(vp.at[sl], tvp.at[sl])
    pltpu.sync_copy(wp.at[sl], twp.at[sl])
    pltpu.sync_copy(vn.at[sl], tvn.at[sl])
    pltpu.sync_copy(wn.at[sl], twn.at[sl])
    pltpu.sync_copy(zeros_hbm.at[sl], acc_num.at[sl])
    pltpu.sync_copy(zeros_hbm.at[sl], acc_dom.at[sl])
    plsc.subcore_barrier()

    buf_a = (idx_a, seg_a, val_a, wt_a)
    buf_b = (idx_b, seg_b, val_b, wt_b)

    def run_edges(adj, vtbl, wtbl):
        base0 = wid * EPW

        def issue_loads(i, bufs):
            idxb, segb, _, _ = bufs
            b = base0 + i * CHUNK
            pltpu.async_copy(adj.at[pl.ds(E_EDGES + b, CHUNK)], idxb, lsem)
            pltpu.async_copy(adj.at[pl.ds(b, CHUNK)], segb, lsem)

        def wait_loads(bufs):
            idxb, segb, _, _ = bufs
            pltpu.make_async_copy(adj.at[pl.ds(0, CHUNK)], idxb, lsem).wait()
            pltpu.make_async_copy(adj.at[pl.ds(0, CHUNK)], segb, lsem).wait()

        def wait_scatters(bufs):
            _, segb, valb, wtb = bufs
            pltpu.make_async_copy(valb, acc_num.at[segb], ssem).wait()
            pltpu.make_async_copy(wtb, acc_dom.at[segb], ssem).wait()

        def gathers(bufs):
            idxb, _, valb, wtb = bufs
            gv = pltpu.async_copy(vtbl.at[idxb], valb, gsem_v)
            gw = pltpu.async_copy(wtbl.at[idxb], wtb, gsem_w)
            return gv, gw

        def scatters(gv, gw, bufs):
            _, segb, valb, wtb = bufs
            gv.wait()
            pltpu.async_copy(valb, acc_num.at[segb], ssem, add=True)
            gw.wait()
            pltpu.async_copy(wtb, acc_dom.at[segb], ssem, add=True)

        issue_loads(0, buf_a)

        def pair(k, carry):
            wait_loads(buf_a)
            gv, gw = gathers(buf_a)

            @pl.when(k >= 1)
            def _():
                wait_scatters(buf_b)

            issue_loads(2 * k + 1, buf_b)
            scatters(gv, gw, buf_a)
            wait_loads(buf_b)
            gv, gw = gathers(buf_b)
            wait_scatters(buf_a)

            @pl.when(k + 1 < NCH // 2)
            def _():
                issue_loads(2 * k + 2, buf_a)

            scatters(gv, gw, buf_b)
            return carry

        lax.fori_loop(0, NCH // 2, pair, 0)
        wait_scatters(buf_b)

    run_edges(adj_pos, tvp, twp)
    run_edges(adj_neg, tvn, twn)
    plsc.subcore_barrier()
    pltpu.sync_copy(acc_num.at[sl], out.at[pl.ds((cid * 2 + 0) * NPAD + sbase, ZPT)])
    pltpu.sync_copy(acc_dom.at[sl], out.at[pl.ds((cid * 2 + 1) * NPAD + sbase, ZPT)])


def _segment_accumulate(vp, wp, vn, wn, zeros, adj_pos, adj_neg):
    mesh = plsc.VectorSubcoreMesh(
        core_axis_name="c", subcore_axis_name="s", num_cores=NC, num_subcores=NS)
    f = pl.kernel(
        _sc_body,
        out_type=jax.ShapeDtypeStruct((NC * 2 * NPAD,), jnp.float32),
        mesh=mesh,
        scratch_types=[
            pltpu.VMEM_SHARED((NPAD,), jnp.float32),
            pltpu.VMEM_SHARED((NPAD,), jnp.float32),
            pltpu.VMEM_SHARED((NPAD,), jnp.float32),
            pltpu.VMEM_SHARED((NPAD,), jnp.float32),
            pltpu.VMEM_SHARED((NPAD,), jnp.float32),
            pltpu.VMEM_SHARED((NPAD,), jnp.float32),
            pltpu.VMEM((CHUNK,), jnp.int32),
            pltpu.VMEM((CHUNK,), jnp.int32),
            pltpu.VMEM((CHUNK,), jnp.float32),
            pltpu.VMEM((CHUNK,), jnp.float32),
            pltpu.VMEM((CHUNK,), jnp.int32),
            pltpu.VMEM((CHUNK,), jnp.int32),
            pltpu.VMEM((CHUNK,), jnp.float32),
            pltpu.VMEM((CHUNK,), jnp.float32),
            pltpu.SemaphoreType.DMA,
            pltpu.SemaphoreType.DMA,
            pltpu.SemaphoreType.DMA,
            pltpu.SemaphoreType.DMA,
        ],
    )
    return f(vp, wp, vn, wn, zeros, adj_pos, adj_neg)


def _loss_body(p_ref, o_ref):
    p = p_ref[...]
    num = p[0, 0] + p[1, 0]
    dom = p[0, 1] + p[1, 1]
    r = num / dom
    sm = 1.0 / (1.0 + jnp.exp(A * (0.5 - r))) + 0.05
    lg = jnp.log(sm)
    row = lax.broadcasted_iota(jnp.int32, (ROWS, 128), 0)
    col = lax.broadcasted_iota(jnp.int32, (ROWS, 128), 1)
    valid = (row * 128 + col) < N_SEG
    o_ref[0, 0] = -jnp.sum(jnp.where(valid, lg, 0.0))


def _finish(parts):
    return pl.pallas_call(
        _loss_body,
        out_shape=jax.ShapeDtypeStruct((1, 1), jnp.float32),
        out_specs=pl.BlockSpec(memory_space=pltpu.SMEM),
    )(parts)


def kernel(xv, adj_pos, adj_neg, is_train):
    xf = xv.reshape(-1).astype(jnp.float32)
    xp = jnp.pad(xf, (0, NPAD - N_NODES)).reshape(ROWS, 128)
    vp, wp, vn, wn = _make_tables(xp)
    zeros = jnp.zeros((NPAD,), jnp.float32)
    parts = _segment_accumulate(
        vp.reshape(NPAD), wp.reshape(NPAD), vn.reshape(NPAD), wn.reshape(NPAD),
        zeros, adj_pos.reshape(-1), adj_neg.reshape(-1))
    loss = _finish(parts.reshape(NC, 2, ROWS, 128))
    return loss[0, 0]
